# baseline (speedup 1.0000x reference)
import jax
import jax.numpy as jnp
from jax import lax
from jax.experimental import pallas as pl
from jax.experimental.pallas import tpu as pltpu
from jax.experimental.pallas import tpu_sc as plsc

_CONF_THRES = 0.25
_IOU_THRES = 0.45
_MAX_ANCHORS = 50

_L = 16
_UNROLL = 8
_NPAD = 5120
_OUTW = 16


def _nms_body(xt_hbm, out_hbm, buf, score, area, outv):
    wid = lax.axis_index("s") * 2 + lax.axis_index("c")

    @pl.when(wid < xt_hbm.shape[0])
    def _():
        b = wid
        pltpu.sync_copy(xt_hbm.at[b], buf)

        lane = lax.iota(jnp.int32, _L)
        ninf = jnp.full((_L,), -jnp.inf, jnp.float32)
        init = (ninf, jnp.full((_L,), jnp.int32(2**30)))

        def acc_update(bv, bp, sc, gidx):
            upd = (sc > bv) | ((sc == bv) & (gidx < bp))
            return jnp.where(upd, sc, bv), jnp.where(upd, gidx, bp)

        def argmax_fin(bv, bp):
            m = jnp.max(bv)
            bi = jnp.min(jnp.where(bv == m, bp, jnp.int32(2**30)))
            return m, bi

        @plsc.parallel_loop(0, _NPAD, _L, unroll=_UNROLL, carry=init)
        def pre_acc(off, carry):
            bv, bp = carry
            sl = pl.ds(off, _L)
            x1 = buf[0, sl]
            y1 = buf[1, sl]
            x2 = buf[2, sl]
            y2 = buf[3, sl]
            cf = buf[4, sl]
            area[sl] = (jnp.maximum(x2 - x1, 0.0)
                        * jnp.maximum(y2 - y1, 0.0))
            sc = jnp.where(cf > _CONF_THRES, cf, -jnp.inf)
            score[sl] = sc
            return acc_update(bv, bp, sc, off + lane)

        m0, bi0 = argmax_fin(*pre_acc)

        def emit(k, m, bi):
            valid = m > -jnp.inf
            iv = jnp.full((_L,), jnp.maximum(bi, 0), jnp.int32)
            r = jnp.zeros((_L,), jnp.int32)
            bx1 = plsc.load_gather(buf, [r, iv])
            by1 = plsc.load_gather(buf, [r + 1, iv])
            bx2 = plsc.load_gather(buf, [r + 2, iv])
            by2 = plsc.load_gather(buf, [r + 3, iv])
            clsv = plsc.load_gather(buf, [r + 5, iv])
            mv = jnp.full((_L,), m, jnp.float32)
            vals = jnp.where(lane == 0, bx1,
                   jnp.where(lane == 1, by1,
                   jnp.where(lane == 2, bx2,
                   jnp.where(lane == 3, by2,
                   jnp.where(lane == 4, mv,
                   jnp.where(lane == 5, clsv, jnp.float32(-1.0)))))))
            vals = jnp.where(valid, vals, jnp.float32(-1.0))
            outv[pl.ds(k * _L, _L)] = vals
            koff = (jnp.minimum(jnp.maximum(bi, 0), _NPAD - 1) // _L) * _L
            ksl = pl.ds(koff, _L)
            score[ksl] = jnp.where(koff + lane == bi, -jnp.inf, score[ksl])
            bx1 = jnp.where(valid, bx1, jnp.float32(0.0))
            by1 = jnp.where(valid, by1, jnp.float32(0.0))
            bx2 = jnp.where(valid, bx2, jnp.float32(-1.0))
            by2 = jnp.where(valid, by2, jnp.float32(-1.0))
            return bx1, by1, bx2, by2

        def keep_step(k, carry):
            m, bi = carry
            bx1, by1, bx2, by2 = emit(k, m, bi)
            area_b = (jnp.maximum(bx2 - bx1, 0.0)
                      * jnp.maximum(by2 - by1, 0.0))

            @plsc.parallel_loop(0, _NPAD, _L, unroll=_UNROLL, carry=init)
            def acc(off, carry):
                bv, bp = carry
                sl = pl.ds(off, _L)
                x1 = buf[0, sl]
                y1 = buf[1, sl]
                x2 = buf[2, sl]
                y2 = buf[3, sl]
                sc = score[sl]
                ar = area[sl]
                ix1 = jnp.maximum(bx1, x1)
                iy1 = jnp.maximum(by1, y1)
                ix2 = jnp.minimum(bx2, x2)
                iy2 = jnp.minimum(by2, y2)
                inter = (jnp.maximum(ix2 - ix1, 0.0)
                         * jnp.maximum(iy2 - iy1, 0.0))
                union = area_b + ar - inter
                iou = inter / (union + 1e-9)
                gidx = off + lane
                sc = jnp.where(iou >= _IOU_THRES, -jnp.inf, sc)
                score[sl] = sc
                return acc_update(bv, bp, sc, gidx)

            return argmax_fin(*acc)

        m, bi = lax.fori_loop(0, _MAX_ANCHORS - 1, keep_step, (m0, bi0))
        emit(_MAX_ANCHORS - 1, m, bi)
        pltpu.sync_copy(outv, out_hbm.at[b])


@jax.jit
def kernel(x):
    B, N, C = x.shape
    pad = jnp.zeros((B, _NPAD - N, C), x.dtype)
    pad = pad.at[:, :, 4].set(-1.0)
    xt = jnp.transpose(jnp.concatenate([x, pad], axis=1), (0, 2, 1))
    xt = jnp.asarray(xt, jnp.float32)

    run = pl.kernel(
        _nms_body,
        out_type=jax.ShapeDtypeStruct((B, _MAX_ANCHORS * _OUTW),
                                      jnp.float32),
        mesh=plsc.VectorSubcoreMesh(core_axis_name="c",
                                    subcore_axis_name="s"),
        scratch_types=[
            pltpu.VMEM((C, _NPAD), jnp.float32),
            pltpu.VMEM((_NPAD,), jnp.float32),
            pltpu.VMEM((_NPAD,), jnp.float32),
            pltpu.VMEM((_MAX_ANCHORS * _OUTW,), jnp.float32),
        ],
        compiler_params=pltpu.CompilerParams(needs_layout_passes=False),
    )
    out = run(xt)
    return out.reshape(B, _MAX_ANCHORS, _OUTW)[:, :, :C]


# speedup vs baseline: 1.4121x; 1.4121x over previous
import jax
import jax.numpy as jnp
from jax import lax
from jax.experimental import pallas as pl
from jax.experimental.pallas import tpu as pltpu
from jax.experimental.pallas import tpu_sc as plsc

_CONF_THRES = 0.25
_IOU_THRES = 0.45
_MAX_ANCHORS = 50

_L = 16
_UNROLL = 4
_NPAD = 5120
_HALF = _NPAD // 2
_OUTW = 16


def _nms_body(xt_hbm, out_hbm, buf, score, area, outv, xbuf, xr, shared):
    c = lax.axis_index("c")
    s = lax.axis_index("s")
    b = c * 8 + lax.rem(s, 8)
    h = s // 8
    base = h * _HALF
    partner = jnp.where(h == 0, s + 8, s - 8)

    pltpu.sync_copy(xt_hbm.at[b], buf)

    lane = lax.iota(jnp.int32, _L)
    ninf = jnp.full((_L,), -jnp.inf, jnp.float32)
    init = (ninf, jnp.full((_L,), jnp.int32(2**30)))

    def acc_update(bv, bp, sc, gidx):
        upd = (sc > bv) | ((sc == bv) & (gidx < bp))
        return jnp.where(upd, sc, bv), jnp.where(upd, gidx, bp)

    def argmax_fin(bv, bp):
        m = jnp.max(bv)
        bi = jnp.min(jnp.where(bv == m, bp, jnp.int32(2**30)))
        return m, bi

    def exchange(m, bi):
        mv = jnp.full((_L,), m, jnp.float32)
        biv = plsc.bitcast(jnp.full((_L,), bi, jnp.int32), jnp.float32)
        xbuf[0, :] = jnp.where(lane < 8, mv, biv)
        pltpu.sync_copy(xbuf.at[0], shared.at[pl.ds(s * _L, _L)])
        plsc.subcore_barrier()
        pltpu.sync_copy(shared, xr)
        rv = plsc.load_gather(xr, [partner * _L + lane])
        plsc.subcore_barrier()
        mo = jnp.max(jnp.where(lane < 8, rv, -jnp.inf))
        ri = plsc.bitcast(rv, jnp.int32)
        bio = jnp.max(jnp.where(lane >= 8, ri, jnp.int32(-2**31 + 1)))
        take = (mo > m) | ((mo == m) & (bio < bi))
        return jnp.where(take, mo, m), jnp.where(take, bio, bi)

    @plsc.parallel_loop(0, _HALF, _L, unroll=_UNROLL, carry=init)
    def pre_acc(off, carry):
        bv, bp = carry
        gsl = pl.ds(base + off, _L)
        sl = pl.ds(off, _L)
        x1 = buf[0, gsl]
        y1 = buf[1, gsl]
        x2 = buf[2, gsl]
        y2 = buf[3, gsl]
        cf = buf[4, gsl]
        area[sl] = (jnp.maximum(x2 - x1, 0.0)
                    * jnp.maximum(y2 - y1, 0.0))
        sc = jnp.where(cf > _CONF_THRES, cf, -jnp.inf)
        score[sl] = sc
        return acc_update(bv, bp, sc, base + off + lane)

    m0, bi0 = exchange(*argmax_fin(*pre_acc))

    def emit(k, m, bi):
        valid = m > -jnp.inf
        iv = jnp.full((_L,), jnp.maximum(bi, 0), jnp.int32)
        r = jnp.zeros((_L,), jnp.int32)
        bx1 = plsc.load_gather(buf, [r, iv])
        by1 = plsc.load_gather(buf, [r + 1, iv])
        bx2 = plsc.load_gather(buf, [r + 2, iv])
        by2 = plsc.load_gather(buf, [r + 3, iv])
        clsv = plsc.load_gather(buf, [r + 5, iv])
        mv = jnp.full((_L,), m, jnp.float32)
        vals = jnp.where(lane == 0, bx1,
               jnp.where(lane == 1, by1,
               jnp.where(lane == 2, bx2,
               jnp.where(lane == 3, by2,
               jnp.where(lane == 4, mv,
               jnp.where(lane == 5, clsv, jnp.float32(-1.0)))))))
        vals = jnp.where(valid, vals, jnp.float32(-1.0))
        outv[pl.ds(k * _L, _L)] = vals
        lbi = bi - base
        koff = (jnp.minimum(jnp.maximum(lbi, 0), _HALF - 1) // _L) * _L
        ksl = pl.ds(koff, _L)
        score[ksl] = jnp.where(koff + lane == lbi, -jnp.inf, score[ksl])
        bx1 = jnp.where(valid, bx1, jnp.float32(0.0))
        by1 = jnp.where(valid, by1, jnp.float32(0.0))
        bx2 = jnp.where(valid, bx2, jnp.float32(-1.0))
        by2 = jnp.where(valid, by2, jnp.float32(-1.0))
        return bx1, by1, bx2, by2

    def keep_step(k, carry):
        m, bi = carry
        bx1, by1, bx2, by2 = emit(k, m, bi)
        area_b = (jnp.maximum(bx2 - bx1, 0.0)
                  * jnp.maximum(by2 - by1, 0.0))

        @plsc.parallel_loop(0, _HALF, _L, unroll=_UNROLL, carry=init)
        def acc(off, carry):
            bv, bp = carry
            gsl = pl.ds(base + off, _L)
            sl = pl.ds(off, _L)
            x1 = buf[0, gsl]
            y1 = buf[1, gsl]
            x2 = buf[2, gsl]
            y2 = buf[3, gsl]
            sc = score[sl]
            ar = area[sl]
            ix1 = jnp.maximum(bx1, x1)
            iy1 = jnp.maximum(by1, y1)
            ix2 = jnp.minimum(bx2, x2)
            iy2 = jnp.minimum(by2, y2)
            inter = (jnp.maximum(ix2 - ix1, 0.0)
                     * jnp.maximum(iy2 - iy1, 0.0))
            union = area_b + ar - inter
            iou = inter / (union + 1e-9)
            sc = jnp.where(iou >= _IOU_THRES, -jnp.inf, sc)
            score[sl] = sc
            return acc_update(bv, bp, sc, base + off + lane)

        return exchange(*argmax_fin(*acc))

    m, bi = lax.fori_loop(0, _MAX_ANCHORS - 1, keep_step, (m0, bi0))
    emit(_MAX_ANCHORS - 1, m, bi)

    @pl.when(h == 0)
    def _():
        pltpu.sync_copy(outv, out_hbm.at[b])


@jax.jit
def kernel(x):
    B, N, C = x.shape
    pad = jnp.zeros((B, _NPAD - N, C), x.dtype)
    pad = pad.at[:, :, 4].set(-1.0)
    xt = jnp.transpose(jnp.concatenate([x, pad], axis=1), (0, 2, 1))
    xt = jnp.asarray(xt, jnp.float32)

    run = pl.kernel(
        _nms_body,
        out_type=jax.ShapeDtypeStruct((B, _MAX_ANCHORS * _OUTW),
                                      jnp.float32),
        mesh=plsc.VectorSubcoreMesh(core_axis_name="c",
                                    subcore_axis_name="s"),
        scratch_types=[
            pltpu.VMEM((C, _NPAD), jnp.float32),
            pltpu.VMEM((_HALF,), jnp.float32),
            pltpu.VMEM((_HALF,), jnp.float32),
            pltpu.VMEM((_MAX_ANCHORS * _OUTW,), jnp.float32),
            pltpu.VMEM((2, _L), jnp.float32),
            pltpu.VMEM((16 * _L,), jnp.float32),
            pltpu.VMEM_SHARED((16 * _L,), jnp.float32),
        ],
        compiler_params=pltpu.CompilerParams(needs_layout_passes=False),
    )
    out = run(xt)
    return out.reshape(B, _MAX_ANCHORS, _OUTW)[:, :, :C]


# speedup vs baseline: 1.4633x; 1.0362x over previous
import jax
import jax.numpy as jnp
from jax import lax
from jax.experimental import pallas as pl
from jax.experimental.pallas import tpu as pltpu
from jax.experimental.pallas import tpu_sc as plsc

_CONF_THRES = 0.25
_IOU_THRES = 0.45
_MAX_ANCHORS = 50

_L = 16
_UNROLL = 4
_NPAD = 5120
_HALF = _NPAD // 2
_OUTW = 16


def _nms_body(xt_hbm, out_hbm, buf, score, area, cx1, cy1, cx2, cy2,
              cidx, outv, xbuf, xr, shared):
    c = lax.axis_index("c")
    s = lax.axis_index("s")
    b = c * 8 + lax.rem(s, 8)
    h = s // 8
    base = h * _HALF
    partner = jnp.where(h == 0, s + 8, s - 8)

    pltpu.sync_copy(xt_hbm.at[b], buf)

    lane = lax.iota(jnp.int32, _L)
    ninf = jnp.full((_L,), -jnp.inf, jnp.float32)
    init = (ninf, jnp.full((_L,), jnp.int32(2**30)))

    def acc_update(bv, bp, sc, gidx):
        upd = (sc > bv) | ((sc == bv) & (gidx < bp))
        return jnp.where(upd, sc, bv), jnp.where(upd, gidx, bp)

    def argmax_fin(bv, bp):
        m = jnp.max(bv)
        bi = jnp.min(jnp.where(bv == m, bp, jnp.int32(2**30)))
        return m, bi

    def exchange(m, bi):
        mv = jnp.full((_L,), m, jnp.float32)
        biv = plsc.bitcast(jnp.full((_L,), bi, jnp.int32), jnp.float32)
        xbuf[0, :] = jnp.where(lane < 8, mv, biv)
        pltpu.sync_copy(xbuf.at[0], shared.at[pl.ds(s * _L, _L)])
        plsc.subcore_barrier()
        pltpu.sync_copy(shared, xr)
        rv = plsc.load_gather(xr, [partner * _L + lane])
        plsc.subcore_barrier()
        mo = jnp.max(jnp.where(lane < 8, rv, -jnp.inf))
        ri = plsc.bitcast(rv, jnp.int32)
        bio = jnp.max(jnp.where(lane >= 8, ri, jnp.int32(-2**31 + 1)))
        take = (mo > m) | ((mo == m) & (bio < bi))
        return jnp.where(take, mo, m), jnp.where(take, bio, bi)

    def pre(ci, carry):
        wofs, bv, bp = carry
        off = ci * _L
        gsl = pl.ds(base + off, _L)
        x1 = buf[0, gsl]
        y1 = buf[1, gsl]
        x2 = buf[2, gsl]
        y2 = buf[3, gsl]
        cf = buf[4, gsl]
        ar = (jnp.maximum(x2 - x1, 0.0)
              * jnp.maximum(y2 - y1, 0.0))
        valid = cf > _CONF_THRES
        gidx = base + off + lane
        wsl = pl.ds(wofs, _L)
        plsc.store_compressed(cx1.at[wsl], x1, mask=valid)
        plsc.store_compressed(cy1.at[wsl], y1, mask=valid)
        plsc.store_compressed(cx2.at[wsl], x2, mask=valid)
        plsc.store_compressed(cy2.at[wsl], y2, mask=valid)
        plsc.store_compressed(area.at[wsl], ar, mask=valid)
        plsc.store_compressed(score.at[wsl], cf, mask=valid)
        plsc.store_compressed(cidx.at[wsl], gidx, mask=valid)
        cnt = jnp.max(plsc.all_reduce_population_count(valid))
        bv, bp = acc_update(bv, bp, jnp.where(valid, cf, -jnp.inf), gidx)
        return wofs + cnt, bv, bp

    wofs, bv, bp = lax.fori_loop(0, _HALF // _L, pre,
                                 (jnp.int32(0), *init))
    tsl = pl.ds(wofs, _L)
    score[tsl] = ninf
    cidx[tsl] = jnp.full((_L,), jnp.int32(2**30))
    cx1[tsl] = jnp.zeros((_L,), jnp.float32)
    cy1[tsl] = jnp.zeros((_L,), jnp.float32)
    cx2[tsl] = jnp.full((_L,), jnp.float32(-1.0))
    cy2[tsl] = jnp.full((_L,), jnp.float32(-1.0))
    area[tsl] = jnp.zeros((_L,), jnp.float32)
    nsw = ((wofs + _L - 1) // _L) * _L

    m0, bi0 = exchange(*argmax_fin(bv, bp))

    def emit(k, m, bi):
        valid = m > -jnp.inf
        safe = jnp.minimum(jnp.maximum(bi, 0), _NPAD - 1)
        iv = jnp.full((_L,), safe, jnp.int32)
        r = jnp.zeros((_L,), jnp.int32)
        bx1 = plsc.load_gather(buf, [r, iv])
        by1 = plsc.load_gather(buf, [r + 1, iv])
        bx2 = plsc.load_gather(buf, [r + 2, iv])
        by2 = plsc.load_gather(buf, [r + 3, iv])
        clsv = plsc.load_gather(buf, [r + 5, iv])
        mv = jnp.full((_L,), m, jnp.float32)
        vals = jnp.where(lane == 0, bx1,
               jnp.where(lane == 1, by1,
               jnp.where(lane == 2, bx2,
               jnp.where(lane == 3, by2,
               jnp.where(lane == 4, mv,
               jnp.where(lane == 5, clsv, jnp.float32(-1.0)))))))
        vals = jnp.where(valid, vals, jnp.float32(-1.0))
        outv[pl.ds(k * _L, _L)] = vals
        bx1 = jnp.where(valid, bx1, jnp.float32(0.0))
        by1 = jnp.where(valid, by1, jnp.float32(0.0))
        bx2 = jnp.where(valid, bx2, jnp.float32(-1.0))
        by2 = jnp.where(valid, by2, jnp.float32(-1.0))
        biv = jnp.where(valid, jnp.full((_L,), bi, jnp.int32), -1)
        return bx1, by1, bx2, by2, biv

    def keep_step(k, carry):
        m, bi = carry
        bx1, by1, bx2, by2, biv = emit(k, m, bi)
        area_b = (jnp.maximum(bx2 - bx1, 0.0)
                  * jnp.maximum(by2 - by1, 0.0))

        @plsc.parallel_loop(0, nsw, _L, unroll=_UNROLL, carry=init)
        def acc(off, carry):
            bv, bp = carry
            sl = pl.ds(off, _L)
            x1 = cx1[sl]
            y1 = cy1[sl]
            x2 = cx2[sl]
            y2 = cy2[sl]
            sc = score[sl]
            ar = area[sl]
            gidx = cidx[sl]
            ix1 = jnp.maximum(bx1, x1)
            iy1 = jnp.maximum(by1, y1)
            ix2 = jnp.minimum(bx2, x2)
            iy2 = jnp.minimum(by2, y2)
            inter = (jnp.maximum(ix2 - ix1, 0.0)
                     * jnp.maximum(iy2 - iy1, 0.0))
            union = area_b + ar - inter
            iou = inter / (union + 1e-9)
            supp = (iou >= _IOU_THRES) | (gidx == biv)
            sc = jnp.where(supp, -jnp.inf, sc)
            score[sl] = sc
            return acc_update(bv, bp, sc, gidx)

        return exchange(*argmax_fin(*acc))

    m, bi = lax.fori_loop(0, _MAX_ANCHORS - 1, keep_step, (m0, bi0))
    emit(_MAX_ANCHORS - 1, m, bi)

    @pl.when(h == 0)
    def _():
        pltpu.sync_copy(outv, out_hbm.at[b])


@jax.jit
def kernel(x):
    B, N, C = x.shape
    pad = jnp.zeros((B, _NPAD - N, C), x.dtype)
    pad = pad.at[:, :, 4].set(-1.0)
    xt = jnp.transpose(jnp.concatenate([x, pad], axis=1), (0, 2, 1))
    xt = jnp.asarray(xt, jnp.float32)

    run = pl.kernel(
        _nms_body,
        out_type=jax.ShapeDtypeStruct((B, _MAX_ANCHORS * _OUTW),
                                      jnp.float32),
        mesh=plsc.VectorSubcoreMesh(core_axis_name="c",
                                    subcore_axis_name="s"),
        scratch_types=[
            pltpu.VMEM((C, _NPAD), jnp.float32),
            pltpu.VMEM((_HALF + _L,), jnp.float32),
            pltpu.VMEM((_HALF + _L,), jnp.float32),
            pltpu.VMEM((_HALF + _L,), jnp.float32),
            pltpu.VMEM((_HALF + _L,), jnp.float32),
            pltpu.VMEM((_HALF + _L,), jnp.float32),
            pltpu.VMEM((_HALF + _L,), jnp.float32),
            pltpu.VMEM((_HALF + _L,), jnp.int32),
            pltpu.VMEM((_MAX_ANCHORS * _OUTW,), jnp.float32),
            pltpu.VMEM((2, _L), jnp.float32),
            pltpu.VMEM((16 * _L,), jnp.float32),
            pltpu.VMEM_SHARED((16 * _L,), jnp.float32),
        ],
        compiler_params=pltpu.CompilerParams(needs_layout_passes=False),
    )
    out = run(xt)
    return out.reshape(B, _MAX_ANCHORS, _OUTW)[:, :, :C]
